# baseline (speedup 1.0000x reference)
import functools

import jax
import jax.numpy as jnp
from jax import lax
from jax.experimental import pallas as pl
from jax.experimental.pallas import tpu as pltpu
from jax.experimental.pallas import tpu_sc as plsc

_THRESHOLD = 64.0
_NEG = -1e30
_LANES = 128
_ROWS = 8
_B = 128
_V = 100000

_NC = 2
_NS = 16
_QCOLS = 49920
_SCV = 2 * _QCOLS
_CH = 4992
_NSEC = _QCOLS // _CH


def _panel_top2_sumexp(x):
    m1 = jnp.max(x, axis=1, keepdims=True)
    eq = x == m1
    runner = jnp.max(jnp.where(eq, _NEG, x), axis=1, keepdims=True)
    cnt = jnp.sum(eq.astype(jnp.float32), axis=1, keepdims=True)
    m2 = jnp.where(cnt > 1.0, m1, runner)
    s = jnp.sum(jnp.exp(x - m1), axis=1, keepdims=True)
    return m1, m2, s


def _merge_top2(a1, a2, b1, b2):
    m1 = jnp.maximum(a1, b1)
    m2 = jnp.maximum(jnp.minimum(a1, b1), jnp.where(a1 >= b1, a2, b2))
    return m1, m2


def _sc_kernel(x_hbm, out_hbm, buf_a, buf_b, m1s, m2s, ss, stage, sem_a,
               sem_b):
    wid = lax.axis_index("s") * _NC + lax.axis_index("c")
    g = wid // 2
    q = wid % 2
    row0 = g * 8
    base = q * _QCOLS
    bufs = (buf_a, buf_b)
    sems = (sem_a, sem_b)
    negv = jnp.full((16,), _NEG, jnp.float32)

    def _start(sec):
        return pltpu.async_copy(
            x_hbm.at[pl.ds(row0, 8), pl.ds(base + sec * _CH, _CH)],
            bufs[sec % 2], sems[sec % 2])

    def _init(r, _):
        m1s[r, :] = negv
        m2s[r, :] = negv
        ss[r, :] = jnp.zeros((16,), jnp.float32)
        return 0

    lax.fori_loop(0, 8, _init, 0)

    handle = _start(0)
    for sec in range(_NSEC):
        handle.wait()
        if sec + 1 < _NSEC:
            handle = _start(sec + 1)
        buf = bufs[sec % 2]

        def _row(r, _):
            def _top2(i, c):
                a10, a20, a11, a21, a12, a22, a13, a23 = c
                o = i * 128
                v0 = buf[r, pl.ds(o, 16)]
                v1 = buf[r, pl.ds(o + 16, 16)]
                v2 = buf[r, pl.ds(o + 32, 16)]
                v3 = buf[r, pl.ds(o + 48, 16)]
                v4 = buf[r, pl.ds(o + 64, 16)]
                v5 = buf[r, pl.ds(o + 80, 16)]
                v6 = buf[r, pl.ds(o + 96, 16)]
                v7 = buf[r, pl.ds(o + 112, 16)]
                a20 = jnp.maximum(a20, jnp.minimum(a10, v0))
                a10 = jnp.maximum(a10, v0)
                a21 = jnp.maximum(a21, jnp.minimum(a11, v1))
                a11 = jnp.maximum(a11, v1)
                a22 = jnp.maximum(a22, jnp.minimum(a12, v2))
                a12 = jnp.maximum(a12, v2)
                a23 = jnp.maximum(a23, jnp.minimum(a13, v3))
                a13 = jnp.maximum(a13, v3)
                a20 = jnp.maximum(a20, jnp.minimum(a10, v4))
                a10 = jnp.maximum(a10, v4)
                a21 = jnp.maximum(a21, jnp.minimum(a11, v5))
                a11 = jnp.maximum(a11, v5)
                a22 = jnp.maximum(a22, jnp.minimum(a12, v6))
                a12 = jnp.maximum(a12, v6)
                a23 = jnp.maximum(a23, jnp.minimum(a13, v7))
                a13 = jnp.maximum(a13, v7)
                return (a10, a20, a11, a21, a12, a22, a13, a23)

            cpairs = lax.fori_loop(0, _CH // 128, _top2, (negv,) * 8)
            b1, b2 = cpairs[0], cpairs[1]
            for u in range(1, 4):
                b1, b2 = _merge_top2(b1, b2, cpairs[2 * u], cpairs[2 * u + 1])
            m1v = m1s[r, :]
            m1n, m2n = _merge_top2(m1v, m2s[r, :], b1, b2)
            m1s[r, :] = m1n
            m2s[r, :] = m2n
            sv = ss[r, :] * jnp.exp(m1v - m1n)

            def _esum(i, c):
                s0, s1, s2, s3 = c
                o = i * 128
                s0 = s0 + jnp.exp(buf[r, pl.ds(o, 16)] - m1n)
                s1 = s1 + jnp.exp(buf[r, pl.ds(o + 16, 16)] - m1n)
                s2 = s2 + jnp.exp(buf[r, pl.ds(o + 32, 16)] - m1n)
                s3 = s3 + jnp.exp(buf[r, pl.ds(o + 48, 16)] - m1n)
                s0 = s0 + jnp.exp(buf[r, pl.ds(o + 64, 16)] - m1n)
                s1 = s1 + jnp.exp(buf[r, pl.ds(o + 80, 16)] - m1n)
                s2 = s2 + jnp.exp(buf[r, pl.ds(o + 96, 16)] - m1n)
                s3 = s3 + jnp.exp(buf[r, pl.ds(o + 112, 16)] - m1n)
                return (s0, s1, s2, s3)

            zs = jnp.zeros((16,), jnp.float32)
            s0, s1, s2, s3 = lax.fori_loop(0, _CH // 128, _esum,
                                           (zs, zs, zs, zs))
            ss[r, :] = sv + (s0 + s1) + (s2 + s3)
            return 0

        lax.fori_loop(0, 8, _row, 0)

    iv = lax.iota(jnp.int32, 16)

    def _fold(r, _):
        m1v = m1s[r, :]
        m2v = m2s[r, :]
        sv = ss[r, :]
        g1 = jnp.max(m1v)
        eq = m1v == g1
        cnt = plsc.all_reduce_population_count(eq)
        runner = jnp.max(jnp.where(eq, _NEG, m1v))
        g2v = jnp.where(cnt > 1, g1, jnp.maximum(runner, jnp.max(m2v)))
        sg = jnp.sum(sv * jnp.exp(m1v - g1))
        ov = jnp.where(iv == 0, g1, jnp.where(iv == 1, g2v,
                                              jnp.where(iv == 2, sg, 0.0)))
        stage[r, :] = ov
        return 0

    lax.fori_loop(0, 8, _fold, 0)
    pltpu.sync_copy(stage, out_hbm.at[q, pl.ds(g * 8, 8), :])


def _tc_tail_kernel(scp_ref, strip_ref, true_ref, res_ref):
    m1, m2, s = _panel_top2_sumexp(strip_ref[...])
    for p in range(2):
        p1 = scp_ref[p, :, 0:1]
        p2 = scp_ref[p, :, 1:2]
        ps = scp_ref[p, :, 2:3]
        n1, n2 = _merge_top2(m1, m2, p1, p2)
        s = s * jnp.exp(m1 - n1) + ps * jnp.exp(p1 - n1)
        m1, m2 = n1, n2

    true = true_ref[...]
    masked_max = jnp.where(true == m1, m2, m1)
    margin = true - masked_max
    lse = m1 + jnp.log(s)
    l = jnp.where(margin >= 0.0, 1.0 - margin, 1.0 - true + lse)
    l = jnp.maximum(l, 0.0)

    ones_row = jnp.ones((1, _LANES), jnp.float32)
    bc = jax.lax.dot_general(l, ones_row, (((1,), (0,)), ((), ())),
                             precision=jax.lax.Precision.HIGHEST)
    br = bc.T
    ii = jax.lax.broadcasted_iota(jnp.int32, (_LANES, _LANES), 0)
    jj = jax.lax.broadcasted_iota(jnp.int32, (_LANES, _LANES), 1)
    prec = ((br < bc) | ((br == bc) & (jj < ii))).astype(jnp.float32)
    incl = jnp.where((br == bc) & (jj == ii), 1.0, prec)
    ones_col = jnp.ones((_LANES, 1), jnp.float32)
    rank = jax.lax.dot_general(prec, ones_col, (((1,), (0,)), ((), ())),
                               precision=jax.lax.Precision.HIGHEST)
    csum = jax.lax.dot_general(incl, l, (((1,), (0,)), ((), ())),
                               precision=jax.lax.Precision.HIGHEST)
    keep = (csum <= _THRESHOLD + 1.0 - rank).astype(jnp.float32)
    c1 = jnp.sum(keep * l)
    c2 = jnp.float32(_LANES) - jnp.sum(keep)
    res_ref[0, 0] = jnp.where(c1 < c2, c2, c1)


@jax.jit
def kernel(output, target):
    B, V = output.shape
    rows = jnp.arange(B, dtype=jnp.int32)
    true = output[rows, target.astype(jnp.int32)].reshape(B, 1)

    sc_fn = pl.kernel(
        _sc_kernel,
        out_type=jax.ShapeDtypeStruct((2, _B, 16), jnp.float32),
        mesh=plsc.VectorSubcoreMesh(core_axis_name="c", subcore_axis_name="s",
                                    num_cores=_NC, num_subcores=_NS),
        scratch_types=[
            pltpu.VMEM((8, _CH), jnp.float32),
            pltpu.VMEM((8, _CH), jnp.float32),
            pltpu.VMEM((8, 16), jnp.float32),
            pltpu.VMEM((8, 16), jnp.float32),
            pltpu.VMEM((8, 16), jnp.float32),
            pltpu.VMEM((8, 16), jnp.float32),
            pltpu.SemaphoreType.DMA,
            pltpu.SemaphoreType.DMA,
        ],
        compiler_params=pltpu.CompilerParams(needs_layout_passes=False),
    )
    scp = sc_fn(output)

    strip = lax.slice(output, (0, _SCV), (B, V))

    res = pl.pallas_call(
        _tc_tail_kernel,
        grid=(1,),
        in_specs=[
            pl.BlockSpec((2, _B, 16), lambda i: (0, 0, 0)),
            pl.BlockSpec((_B, V - _SCV), lambda i: (0, 0)),
            pl.BlockSpec((_B, 1), lambda i: (0, 0)),
        ],
        out_specs=pl.BlockSpec((1, 1), lambda i: (0, 0),
                               memory_space=pltpu.SMEM),
        out_shape=jax.ShapeDtypeStruct((1, 1), jnp.float32),
    )(scp, strip, true)
    return res[0, 0]


# speedup vs baseline: 1.4409x; 1.4409x over previous
import functools

import jax
import jax.numpy as jnp
from jax.experimental import pallas as pl
from jax.experimental.pallas import tpu as pltpu

_THRESHOLD = 64.0
_NEG = -1e30
_LANES = 128


def _block_top2(read_chunk, nchunks):
    bm1 = read_chunk(0)
    bm2 = jnp.full_like(bm1, _NEG)
    for k in range(1, nchunks):
        xk = read_chunk(k)
        bm2 = jnp.maximum(bm2, jnp.minimum(bm1, xk))
        bm1 = jnp.maximum(bm1, xk)
    return bm1, bm2


def _merge_top2(a1, a2, b1, b2):
    m1 = jnp.maximum(a1, b1)
    m2 = jnp.maximum(jnp.minimum(a1, b1), jnp.where(a1 >= b1, a2, b2))
    return m1, m2


def _tight_closs_kernel(out_mat, true_ref, res_ref, m1_ref, m2_ref, s_ref,
                        *, blk, ncols, nblocks):
    j = pl.program_id(0)
    nchunks = blk // _LANES

    @pl.when(j == 0)
    def _init():
        m1_ref[...] = jnp.full_like(m1_ref, _NEG)
        m2_ref[...] = jnp.full_like(m2_ref, _NEG)
        s_ref[...] = jnp.zeros_like(s_ref)

    def _process(read_chunk):
        bm1, bm2 = _block_top2(read_chunk, nchunks)
        a1, a2 = m1_ref[...], m2_ref[...]
        m1n, m2n = _merge_top2(a1, a2, bm1, bm2)
        m1_ref[...] = m1n
        m2_ref[...] = m2n
        es = s_ref[...] * jnp.exp(a1 - m1n)
        for k in range(nchunks):
            es = es + jnp.exp(read_chunk(k) - m1n)
        s_ref[...] = es

    @pl.when(j < nblocks - 1)
    def _steady():
        _process(lambda k: out_mat[:, k * _LANES:(k + 1) * _LANES])

    @pl.when(j == nblocks - 1)
    def _last():
        base = j * blk
        civ = jax.lax.broadcasted_iota(jnp.int32, (128, _LANES), 1)

        def _read_masked(k):
            xk = out_mat[:, k * _LANES:(k + 1) * _LANES]
            return jnp.where(base + k * _LANES + civ < ncols, xk, _NEG)

        _process(_read_masked)

        m1, m2 = m1_ref[...], m2_ref[...]
        sh = _LANES
        while sh > 1:
            sh //= 2
            b1 = pltpu.roll(m1, sh, 1)
            b2 = pltpu.roll(m2, sh, 1)
            m1, m2 = _merge_top2(m1, m2, b1, b2)
        row_m1 = jnp.max(m1_ref[...], axis=1, keepdims=True)
        row_m2 = m2[:, 0:1]
        s = s_ref[...]
        row_s = jnp.sum(s * jnp.exp(m1_ref[...] - row_m1), axis=1,
                        keepdims=True)

        true = true_ref[...]
        masked_max = jnp.where(true == row_m1, row_m2, row_m1)
        margin = true - masked_max
        lse = row_m1 + jnp.log(row_s)
        l = jnp.where(margin >= 0.0, 1.0 - margin, 1.0 - true + lse)
        l = jnp.maximum(l, 0.0)

        ones_row = jnp.ones((1, _LANES), jnp.float32)
        bc = jax.lax.dot_general(l, ones_row, (((1,), (0,)), ((), ())),
                                 precision=jax.lax.Precision.HIGHEST)
        br = bc.T
        ii = jax.lax.broadcasted_iota(jnp.int32, (_LANES, _LANES), 0)
        jj = jax.lax.broadcasted_iota(jnp.int32, (_LANES, _LANES), 1)
        prec = ((br < bc) | ((br == bc) & (jj < ii))).astype(jnp.float32)
        incl = jnp.where((br == bc) & (jj == ii), 1.0, prec)
        ones_col = jnp.ones((_LANES, 1), jnp.float32)
        rank = jax.lax.dot_general(prec, ones_col, (((1,), (0,)), ((), ())),
                                   precision=jax.lax.Precision.HIGHEST)
        csum = jax.lax.dot_general(incl, l, (((1,), (0,)), ((), ())),
                                   precision=jax.lax.Precision.HIGHEST)
        keep = (csum <= _THRESHOLD + 1.0 - rank).astype(jnp.float32)
        c1 = jnp.sum(keep * l)
        c2 = jnp.float32(_LANES) - jnp.sum(keep)
        res_ref[0, 0] = jnp.where(c1 < c2, c2, c1)


@jax.jit
def kernel(output, target):
    B, V = output.shape
    blk = 16384
    nblocks = pl.cdiv(V, blk)
    rows = jnp.arange(B, dtype=jnp.int32)
    true = output[rows, target.astype(jnp.int32)].reshape(B, 1)

    res = pl.pallas_call(
        functools.partial(_tight_closs_kernel, blk=blk, ncols=V,
                          nblocks=nblocks),
        grid=(nblocks,),
        in_specs=[
            pl.BlockSpec((B, blk), lambda j: (0, j)),
            pl.BlockSpec((B, 1), lambda j: (0, 0)),
        ],
        out_specs=pl.BlockSpec((1, 1), lambda j: (0, 0),
                               memory_space=pltpu.SMEM),
        out_shape=jax.ShapeDtypeStruct((1, 1), jnp.float32),
        scratch_shapes=[
            pltpu.VMEM((B, _LANES), jnp.float32),
            pltpu.VMEM((B, _LANES), jnp.float32),
            pltpu.VMEM((B, _LANES), jnp.float32),
        ],
    )(output, true)
    return res[0, 0]


# speedup vs baseline: 1.4630x; 1.0153x over previous
import functools

import jax
import jax.numpy as jnp
from jax.experimental import pallas as pl
from jax.experimental.pallas import tpu as pltpu

_THRESHOLD = 64.0
_NEG = -1e30
_LANES = 128


def _block_top2(read_chunk, nchunks):
    bm1 = read_chunk(0)
    bm2 = jnp.full_like(bm1, _NEG)
    for k in range(1, nchunks):
        xk = read_chunk(k)
        bm2 = jnp.maximum(bm2, jnp.minimum(bm1, xk))
        bm1 = jnp.maximum(bm1, xk)
    return bm1, bm2


def _merge_top2(a1, a2, b1, b2):
    m1 = jnp.maximum(a1, b1)
    m2 = jnp.maximum(jnp.minimum(a1, b1), jnp.where(a1 >= b1, a2, b2))
    return m1, m2


def _tight_closs_kernel(xa_ref, xb_ref, true_ref, res_ref, m1_ref, m2_ref,
                        s_ref, *, blk, ncols, nsteps):
    j = pl.program_id(0)
    nchunks = blk // _LANES

    @pl.when(j == 0)
    def _init():
        m1_ref[...] = jnp.full_like(m1_ref, _NEG)
        m2_ref[...] = jnp.full_like(m2_ref, _NEG)
        s_ref[...] = jnp.zeros_like(s_ref)

    def _process(read_chunk):
        bm1, bm2 = _block_top2(read_chunk, nchunks)
        a1, a2 = m1_ref[...], m2_ref[...]
        m1n, m2n = _merge_top2(a1, a2, bm1, bm2)
        m1_ref[...] = m1n
        m2_ref[...] = m2n
        es = s_ref[...] * jnp.exp(a1 - m1n)
        for k in range(nchunks):
            es = es + jnp.exp(read_chunk(k) - m1n)
        s_ref[...] = es

    @pl.when(j < nsteps - 1)
    def _steady():
        _process(lambda k: xa_ref[:, k * _LANES:(k + 1) * _LANES])
        _process(lambda k: xb_ref[:, k * _LANES:(k + 1) * _LANES])

    @pl.when(j == nsteps - 1)
    def _last():
        base = j * 2 * blk
        civ = jax.lax.broadcasted_iota(jnp.int32, (128, _LANES), 1)

        def _read_masked(k):
            xk = xa_ref[:, k * _LANES:(k + 1) * _LANES]
            return jnp.where(base + k * _LANES + civ < ncols, xk, _NEG)

        _process(_read_masked)

        m1, m2 = m1_ref[...], m2_ref[...]
        sh = _LANES
        while sh > 1:
            sh //= 2
            b1 = pltpu.roll(m1, sh, 1)
            b2 = pltpu.roll(m2, sh, 1)
            m1, m2 = _merge_top2(m1, m2, b1, b2)
        row_m1 = jnp.max(m1_ref[...], axis=1, keepdims=True)
        row_m2 = m2[:, 0:1]
        s = s_ref[...]
        row_s = jnp.sum(s * jnp.exp(m1_ref[...] - row_m1), axis=1,
                        keepdims=True)

        true = true_ref[...]
        masked_max = jnp.where(true == row_m1, row_m2, row_m1)
        margin = true - masked_max
        lse = row_m1 + jnp.log(row_s)
        l = jnp.where(margin >= 0.0, 1.0 - margin, 1.0 - true + lse)
        l = jnp.maximum(l, 0.0)

        ones_row = jnp.ones((1, _LANES), jnp.float32)
        bc = jax.lax.dot_general(l, ones_row, (((1,), (0,)), ((), ())),
                                 precision=jax.lax.Precision.HIGHEST)
        br = bc.T
        ii = jax.lax.broadcasted_iota(jnp.int32, (_LANES, _LANES), 0)
        jj = jax.lax.broadcasted_iota(jnp.int32, (_LANES, _LANES), 1)
        prec = ((br < bc) | ((br == bc) & (jj < ii))).astype(jnp.float32)
        incl = jnp.where((br == bc) & (jj == ii), 1.0, prec)
        ones_col = jnp.ones((_LANES, 1), jnp.float32)
        rank = jax.lax.dot_general(prec, ones_col, (((1,), (0,)), ((), ())),
                                   precision=jax.lax.Precision.HIGHEST)
        csum = jax.lax.dot_general(incl, l, (((1,), (0,)), ((), ())),
                                   precision=jax.lax.Precision.HIGHEST)
        keep = (csum <= _THRESHOLD + 1.0 - rank).astype(jnp.float32)
        c1 = jnp.sum(keep * l)
        c2 = jnp.float32(_LANES) - jnp.sum(keep)
        res_ref[0, 0] = jnp.where(c1 < c2, c2, c1)


@jax.jit
def kernel(output, target):
    B, V = output.shape
    blk = 8192
    nsteps = pl.cdiv(V, 2 * blk)
    nblk = pl.cdiv(V, blk)
    rows = jnp.arange(B, dtype=jnp.int32)
    true = output[rows, target.astype(jnp.int32)].reshape(B, 1)

    res = pl.pallas_call(
        functools.partial(_tight_closs_kernel, blk=blk, ncols=V,
                          nsteps=nsteps),
        grid=(nsteps,),
        in_specs=[
            pl.BlockSpec((B, blk), lambda j: (0, 2 * j)),
            pl.BlockSpec((B, blk),
                         lambda j: (0, jnp.minimum(2 * j + 1, nblk - 1))),
            pl.BlockSpec((B, 1), lambda j: (0, 0)),
        ],
        out_specs=pl.BlockSpec((1, 1), lambda j: (0, 0),
                               memory_space=pltpu.SMEM),
        out_shape=jax.ShapeDtypeStruct((1, 1), jnp.float32),
        scratch_shapes=[
            pltpu.VMEM((B, _LANES), jnp.float32),
            pltpu.VMEM((B, _LANES), jnp.float32),
            pltpu.VMEM((B, _LANES), jnp.float32),
        ],
    )(output, output, true)
    return res[0, 0]
